# baseline (speedup 1.0000x reference)
import functools

import jax
import jax.numpy as jnp
from jax import lax
from jax.experimental import pallas as pl
from jax.experimental.pallas import tpu as pltpu
from jax.experimental.pallas import tpu_sc as plsc

N_NODES = 50000
N_EDGES = 800000

NW = 32
N_PAD = 51200
E_PAD = 819200
E_PER = E_PAD // NW
CHUNK = 6400
N_CHUNKS = E_PER // CHUNK
UNROLL = 16
GROUPS = CHUNK // (16 * UNROLL)

_MESH = plsc.VectorSubcoreMesh(core_axis_name="c", subcore_axis_name="s")
_SC_PARAMS = pltpu.CompilerParams(needs_layout_passes=False)


def _zero_acc(acc_v):
    zv = jnp.zeros((16,), jnp.float32)

    def body(i, _):
        o = i * 128
        for k in range(8):
            acc_v[pl.ds(o + k * 16, 16)] = zv
        return 0
    lax.fori_loop(0, N_PAD // 128, body, 0)


@functools.partial(
    pl.kernel,
    out_type=jax.ShapeDtypeStruct((NW, N_PAD), jnp.float32),
    mesh=_MESH,
    compiler_params=_SC_PARAMS,
    scratch_types=[
        pltpu.VMEM((E_PER,), jnp.int32),
        pltpu.VMEM((E_PER,), jnp.float32),
        pltpu.VMEM((N_PAD,), jnp.float32),
    ],
)
def _sc_deg(dst_h, w_h, out_h, dst_v, w_v, acc_v):
    wid = lax.axis_index("s") * 2 + lax.axis_index("c")
    base = wid * E_PER
    pltpu.sync_copy(dst_h.at[pl.ds(base, E_PER)], dst_v)
    pltpu.sync_copy(w_h.at[pl.ds(base, E_PER)], w_v)
    _zero_acc(acc_v)

    @plsc.parallel_loop(0, E_PER, step=16, unroll=UNROLL)
    def g_body(o):
        dv = dst_v[pl.ds(o, 16)]
        wv = w_v[pl.ds(o, 16)]
        plsc.addupdate_scatter(acc_v, [dv], wv)
    pltpu.sync_copy(acc_v, out_h.at[wid])


def _edge_phase(src_h, dst_h, w_h, base, src_v, dst_v, w_v, tab_v, acc_v):
    def chunk_body(ci, _):
        off = base + ci * CHUNK
        pltpu.sync_copy(src_h.at[pl.ds(off, CHUNK)], src_v)
        pltpu.sync_copy(dst_h.at[pl.ds(off, CHUNK)], dst_v)
        pltpu.sync_copy(w_h.at[pl.ds(off, CHUNK)], w_v)

        @plsc.parallel_loop(0, CHUNK, step=16, unroll=UNROLL)
        def g_body(o):
            sv = src_v[pl.ds(o, 16)]
            dv = dst_v[pl.ds(o, 16)]
            wv = w_v[pl.ds(o, 16)]
            yv = plsc.load_gather(tab_v, [sv])
            plsc.addupdate_scatter(acc_v, [dv], wv * yv)
        return 0
    lax.fori_loop(0, N_CHUNKS, chunk_body, 0)


@functools.partial(
    pl.kernel,
    out_type=jax.ShapeDtypeStruct((2, NW, N_PAD), jnp.float32),
    mesh=_MESH,
    compiler_params=_SC_PARAMS,
    scratch_types=[
        pltpu.VMEM((CHUNK,), jnp.int32),
        pltpu.VMEM((CHUNK,), jnp.int32),
        pltpu.VMEM((CHUNK,), jnp.float32),
        pltpu.VMEM((N_PAD,), jnp.float32),
        pltpu.VMEM((N_PAD,), jnp.float32),
    ],
)
def _sc_s_pass(src_h, dst_h, w_h, y0_h, y1_h, out_h,
               src_v, dst_v, w_v, tab_v, acc_v):
    wid = lax.axis_index("s") * 2 + lax.axis_index("c")
    base = wid * E_PER
    for ph, tab_h in enumerate((y0_h, y1_h)):
        pltpu.sync_copy(tab_h, tab_v)
        _zero_acc(acc_v)
        _edge_phase(src_h, dst_h, w_h, base, src_v, dst_v, w_v, tab_v, acc_v)
        pltpu.sync_copy(acc_v, out_h.at[ph, wid])


@functools.partial(
    pl.kernel,
    out_type=jax.ShapeDtypeStruct((NW, N_PAD), jnp.float32),
    mesh=_MESH,
    compiler_params=_SC_PARAMS,
    scratch_types=[
        pltpu.VMEM((CHUNK,), jnp.int32),
        pltpu.VMEM((CHUNK,), jnp.int32),
        pltpu.VMEM((CHUNK,), jnp.float32),
        pltpu.VMEM((N_PAD,), jnp.float32),
        pltpu.VMEM((N_PAD,), jnp.float32),
    ],
)
def _sc_gather_scatter(src_h, dst_h, w_h, tab_h, out_h,
                       src_v, dst_v, w_v, tab_v, acc_v):
    wid = lax.axis_index("s") * 2 + lax.axis_index("c")
    base = wid * E_PER
    pltpu.sync_copy(tab_h, tab_v)
    _zero_acc(acc_v)
    _edge_phase(src_h, dst_h, w_h, base, src_v, dst_v, w_v, tab_v, acc_v)
    pltpu.sync_copy(acc_v, out_h.at[wid])


def _bfr(a):
    return a.astype(jnp.bfloat16).astype(jnp.float32)


def _k2_body(degp_ref, x0_ref, x1_ref, dis_ref, y0_ref, y1_ref):
    deg = jnp.sum(degp_ref[...], axis=0) + 1.0
    dis = jnp.where(deg > 0, 1.0 / jnp.sqrt(deg), 0.0)
    dis_ref[...] = dis
    y0_ref[...] = dis * _bfr(x0_ref[...])
    y1_ref[...] = dis * _bfr(x1_ref[...])


def _tc_k2(degp, x0, x1):
    return pl.pallas_call(
        _k2_body,
        out_shape=[jax.ShapeDtypeStruct((N_PAD,), jnp.float32)] * 3,
    )(degp, x0, x1)


_BN = 5120


def _k4_body(s0p, s1p, disr, x0r, x1r, w1r, b1r, w2r, zr, ur):
    S0 = jnp.sum(s0p[...], axis=0)
    S1 = jnp.sum(s1p[...], axis=0)
    dis = disr[...]
    d2 = dis * dis
    c0 = dis * S0 + d2 * _bfr(x0r[...])
    c1 = dis * S1 + d2 * _bfr(x1r[...])
    w1b = _bfr(w1r[...])
    ht = (w1b[:, 0:1] * c0[None, :] + w1b[:, 1:2] * c1[None, :]
          + b1r[...])
    ht = jnp.where(ht > 0, ht, jnp.exp(jnp.minimum(ht, 0.0)) - 1.0)
    z = jnp.sum(_bfr(ht) * _bfr(w2r[...]), axis=0)
    zr[...] = z
    ur[...] = dis * z


def _tc_k4(s0p, s1p, dis, x0, x1, W1, b1, W2m):
    node = pl.BlockSpec((_BN,), lambda i: (i,))
    part = pl.BlockSpec((NW, _BN), lambda i: (0, i))
    return pl.pallas_call(
        _k4_body,
        grid=(N_PAD // _BN,),
        in_specs=[
            part, part, node, node, node,
            pl.BlockSpec((128, 2), lambda i: (0, 0)),
            pl.BlockSpec((128, 1), lambda i: (0, 0)),
            pl.BlockSpec((128, 1), lambda i: (0, 0)),
        ],
        out_specs=[node, node],
        out_shape=[jax.ShapeDtypeStruct((N_PAD,), jnp.float32)] * 2,
    )(s0p, s1p, dis, x0, x1, W1, b1, W2m)


def _k6_body(tp, disr, zr, b2r, outr):
    T = jnp.sum(tp[...], axis=0)
    dis = disr[...]
    outr[...] = dis * T + dis * dis * zr[...] + b2r[0]


def _tc_k6(tp, dis, z, b2):
    return pl.pallas_call(
        _k6_body,
        in_specs=[
            pl.BlockSpec(memory_space=pltpu.VMEM),
            pl.BlockSpec(memory_space=pltpu.VMEM),
            pl.BlockSpec(memory_space=pltpu.VMEM),
            pl.BlockSpec(memory_space=pltpu.SMEM),
        ],
        out_shape=jax.ShapeDtypeStruct((N_PAD,), jnp.float32),
    )(tp, dis, z, b2)


def kernel(x, edge_index, edge_weights, W1, b1, W2, b2):
    src = edge_index[0].astype(jnp.int32)
    dst = edge_index[1].astype(jnp.int32)
    w = edge_weights[:, 0]
    epad = E_PAD - N_EDGES
    srcp = jnp.concatenate([src, jnp.zeros((epad,), jnp.int32)])
    dstp = jnp.concatenate([dst, jnp.zeros((epad,), jnp.int32)])
    wp = jnp.concatenate([w, jnp.zeros((epad,), jnp.float32)])
    npad = N_PAD - N_NODES
    x0 = jnp.concatenate([x[:, 0], jnp.zeros((npad,), jnp.float32)])
    x1 = jnp.concatenate([x[:, 1], jnp.zeros((npad,), jnp.float32)])

    degp = _sc_deg(dstp, wp)
    dis, y0, y1 = _tc_k2(degp, x0, x1)
    sp = _sc_s_pass(srcp, dstp, wp, y0, y1)
    z, u = _tc_k4(sp[0], sp[1], dis, x0, x1, W1.T, b1[:, None], W2)
    tp = _sc_gather_scatter(srcp, dstp, wp, u)
    out = _tc_k6(tp, dis, z, b2)
    return out[:N_NODES]


# speedup vs baseline: 1.0724x; 1.0724x over previous
import functools

import jax
import jax.numpy as jnp
from jax import lax
from jax.experimental import pallas as pl
from jax.experimental.pallas import tpu as pltpu
from jax.experimental.pallas import tpu_sc as plsc

N_NODES = 50000
N_EDGES = 800000

NW = 32
N_PAD = 51200
E_PAD = 819200
E_PER = E_PAD // NW
CHUNK = 3200
N_CHUNKS = E_PER // CHUNK
UNROLL = 8
GROUPS = CHUNK // (16 * UNROLL)

_MESH = plsc.VectorSubcoreMesh(core_axis_name="c", subcore_axis_name="s")
_SC_PARAMS = pltpu.CompilerParams(needs_layout_passes=False)


def _zero_acc(acc_v):
    zv = jnp.zeros((16,), jnp.float32)

    def body(i, _):
        o = i * 128
        for k in range(8):
            acc_v[pl.ds(o + k * 16, 16)] = zv
        return 0
    lax.fori_loop(0, N_PAD // 128, body, 0)


@functools.partial(
    pl.kernel,
    out_type=jax.ShapeDtypeStruct((NW, N_PAD), jnp.float32),
    mesh=_MESH,
    compiler_params=_SC_PARAMS,
    scratch_types=[
        pltpu.VMEM((E_PER,), jnp.int32),
        pltpu.VMEM((E_PER,), jnp.float32),
        pltpu.VMEM((N_PAD,), jnp.float32),
        pltpu.SemaphoreType.DMA,
    ],
)
def _sc_deg(dst_h, w_h, out_h, dst_v, w_v, acc_v, sem):
    wid = lax.axis_index("s") * 2 + lax.axis_index("c")
    base = wid * E_PER
    d1 = pltpu.async_copy(dst_h.at[pl.ds(base, E_PER)], dst_v, sem)
    d2 = pltpu.async_copy(w_h.at[pl.ds(base, E_PER)], w_v, sem)
    _zero_acc(acc_v)
    d1.wait()
    d2.wait()

    @plsc.parallel_loop(0, E_PER, step=16, unroll=UNROLL)
    def g_body(o):
        dv = dst_v[pl.ds(o, 16)]
        wv = w_v[pl.ds(o, 16)]
        plsc.addupdate_scatter(acc_v, [dv], wv)
    pltpu.sync_copy(acc_v, out_h.at[wid])


def _edge_phase(src_h, dst_h, w_h, base, src_v, dst_v, w_v, tab_v, acc_v,
                sem0, sem1):
    sems = (sem0, sem1)

    def start(ci):
        off = base + ci * CHUNK
        s = sems[ci % 2]
        b = ci % 2
        return (
            pltpu.async_copy(src_h.at[pl.ds(off, CHUNK)], src_v.at[b], s),
            pltpu.async_copy(dst_h.at[pl.ds(off, CHUNK)], dst_v.at[b], s),
            pltpu.async_copy(w_h.at[pl.ds(off, CHUNK)], w_v.at[b], s),
        )

    descs = start(0)
    for ci in range(N_CHUNKS):
        b = ci % 2
        nxt = start(ci + 1) if ci + 1 < N_CHUNKS else None
        for d in descs:
            d.wait()

        @plsc.parallel_loop(0, CHUNK, step=16, unroll=UNROLL)
        def g_body(o):
            sv = src_v[b, pl.ds(o, 16)]
            dv = dst_v[b, pl.ds(o, 16)]
            wv = w_v[b, pl.ds(o, 16)]
            yv = plsc.load_gather(tab_v, [sv])
            plsc.addupdate_scatter(acc_v, [dv], wv * yv)
        descs = nxt


@functools.partial(
    pl.kernel,
    out_type=jax.ShapeDtypeStruct((2, NW, N_PAD), jnp.float32),
    mesh=_MESH,
    compiler_params=_SC_PARAMS,
    scratch_types=[
        pltpu.VMEM((2, CHUNK), jnp.int32),
        pltpu.VMEM((2, CHUNK), jnp.int32),
        pltpu.VMEM((2, CHUNK), jnp.float32),
        pltpu.VMEM((N_PAD,), jnp.float32),
        pltpu.VMEM((N_PAD,), jnp.float32),
        pltpu.SemaphoreType.DMA,
        pltpu.SemaphoreType.DMA,
    ],
)
def _sc_s_pass(src_h, dst_h, w_h, y0_h, y1_h, out_h,
               src_v, dst_v, w_v, tab_v, acc_v, sem0, sem1):
    wid = lax.axis_index("s") * 2 + lax.axis_index("c")
    base = wid * E_PER
    for ph, tab_h in enumerate((y0_h, y1_h)):
        pltpu.sync_copy(tab_h, tab_v)
        _zero_acc(acc_v)
        _edge_phase(src_h, dst_h, w_h, base, src_v, dst_v, w_v, tab_v, acc_v,
                    sem0, sem1)
        pltpu.sync_copy(acc_v, out_h.at[ph, wid])


@functools.partial(
    pl.kernel,
    out_type=jax.ShapeDtypeStruct((NW, N_PAD), jnp.float32),
    mesh=_MESH,
    compiler_params=_SC_PARAMS,
    scratch_types=[
        pltpu.VMEM((2, CHUNK), jnp.int32),
        pltpu.VMEM((2, CHUNK), jnp.int32),
        pltpu.VMEM((2, CHUNK), jnp.float32),
        pltpu.VMEM((N_PAD,), jnp.float32),
        pltpu.VMEM((N_PAD,), jnp.float32),
        pltpu.SemaphoreType.DMA,
        pltpu.SemaphoreType.DMA,
    ],
)
def _sc_gather_scatter(src_h, dst_h, w_h, tab_h, out_h,
                       src_v, dst_v, w_v, tab_v, acc_v, sem0, sem1):
    wid = lax.axis_index("s") * 2 + lax.axis_index("c")
    base = wid * E_PER
    pltpu.sync_copy(tab_h, tab_v)
    _zero_acc(acc_v)
    _edge_phase(src_h, dst_h, w_h, base, src_v, dst_v, w_v, tab_v, acc_v,
                sem0, sem1)
    pltpu.sync_copy(acc_v, out_h.at[wid])


def _bfr(a):
    return a.astype(jnp.bfloat16).astype(jnp.float32)


def _k2_body(degp_ref, x0_ref, x1_ref, dis_ref, y0_ref, y1_ref):
    deg = jnp.sum(degp_ref[...], axis=0) + 1.0
    dis = jnp.where(deg > 0, 1.0 / jnp.sqrt(deg), 0.0)
    dis_ref[...] = dis
    y0_ref[...] = dis * _bfr(x0_ref[...])
    y1_ref[...] = dis * _bfr(x1_ref[...])


def _tc_k2(degp, x0, x1):
    return pl.pallas_call(
        _k2_body,
        out_shape=[jax.ShapeDtypeStruct((N_PAD,), jnp.float32)] * 3,
    )(degp, x0, x1)


_BN = 5120


def _k4_body(s0p, s1p, disr, x0r, x1r, w1r, b1r, w2r, zr, ur):
    S0 = jnp.sum(s0p[...], axis=0)
    S1 = jnp.sum(s1p[...], axis=0)
    dis = disr[...]
    d2 = dis * dis
    c0 = dis * S0 + d2 * _bfr(x0r[...])
    c1 = dis * S1 + d2 * _bfr(x1r[...])
    w1b = _bfr(w1r[...])
    ht = (w1b[:, 0:1] * c0[None, :] + w1b[:, 1:2] * c1[None, :]
          + b1r[...])
    ht = jnp.where(ht > 0, ht, jnp.exp(jnp.minimum(ht, 0.0)) - 1.0)
    z = jnp.sum(_bfr(ht) * _bfr(w2r[...]), axis=0)
    zr[...] = z
    ur[...] = dis * z


def _tc_k4(s0p, s1p, dis, x0, x1, W1, b1, W2m):
    node = pl.BlockSpec((_BN,), lambda i: (i,))
    part = pl.BlockSpec((NW, _BN), lambda i: (0, i))
    return pl.pallas_call(
        _k4_body,
        grid=(N_PAD // _BN,),
        in_specs=[
            part, part, node, node, node,
            pl.BlockSpec((128, 2), lambda i: (0, 0)),
            pl.BlockSpec((128, 1), lambda i: (0, 0)),
            pl.BlockSpec((128, 1), lambda i: (0, 0)),
        ],
        out_specs=[node, node],
        out_shape=[jax.ShapeDtypeStruct((N_PAD,), jnp.float32)] * 2,
    )(s0p, s1p, dis, x0, x1, W1, b1, W2m)


def _k6_body(tp, disr, zr, b2r, outr):
    T = jnp.sum(tp[...], axis=0)
    dis = disr[...]
    outr[...] = dis * T + dis * dis * zr[...] + b2r[0]


def _tc_k6(tp, dis, z, b2):
    return pl.pallas_call(
        _k6_body,
        in_specs=[
            pl.BlockSpec(memory_space=pltpu.VMEM),
            pl.BlockSpec(memory_space=pltpu.VMEM),
            pl.BlockSpec(memory_space=pltpu.VMEM),
            pl.BlockSpec(memory_space=pltpu.SMEM),
        ],
        out_shape=jax.ShapeDtypeStruct((N_PAD,), jnp.float32),
    )(tp, dis, z, b2)


def kernel(x, edge_index, edge_weights, W1, b1, W2, b2):
    src = edge_index[0].astype(jnp.int32)
    dst = edge_index[1].astype(jnp.int32)
    w = edge_weights[:, 0]
    epad = E_PAD - N_EDGES
    srcp = jnp.concatenate([src, jnp.zeros((epad,), jnp.int32)])
    dstp = jnp.concatenate([dst, jnp.zeros((epad,), jnp.int32)])
    wp = jnp.concatenate([w, jnp.zeros((epad,), jnp.float32)])
    npad = N_PAD - N_NODES
    x0 = jnp.concatenate([x[:, 0], jnp.zeros((npad,), jnp.float32)])
    x1 = jnp.concatenate([x[:, 1], jnp.zeros((npad,), jnp.float32)])

    degp = _sc_deg(dstp, wp)
    dis, y0, y1 = _tc_k2(degp, x0, x1)
    sp = _sc_s_pass(srcp, dstp, wp, y0, y1)
    z, u = _tc_k4(sp[0], sp[1], dis, x0, x1, W1.T, b1[:, None], W2)
    tp = _sc_gather_scatter(srcp, dstp, wp, u)
    out = _tc_k6(tp, dis, z, b2)
    return out[:N_NODES]
